# baseline (speedup 1.0000x reference)
import functools

import jax
import jax.numpy as jnp
from jax import lax
from jax.experimental import pallas as pl
from jax.experimental.pallas import tpu as pltpu
from jax.experimental.pallas import tpu_sc as plsc

NUM_ROWS = 1000000
BATCH = 16384
EMBED_DIM = 16
VEC = 16
ICHUNK = 128


@jax.jit
def _gmf(user, item, user_table, item_table):
    info = plsc.get_sparse_core_info()
    nc, ns = info.num_cores, info.num_subcores
    nw = nc * ns
    b_per_w = BATCH // nw
    n_chunks = b_per_w // ICHUNK

    ut1 = user_table.T.reshape(NUM_ROWS * EMBED_DIM)
    it1 = item_table.T.reshape(NUM_ROWS * EMBED_DIM)

    mesh = plsc.VectorSubcoreMesh(core_axis_name="c", subcore_axis_name="s")

    @functools.partial(
        pl.kernel,
        out_type=jax.ShapeDtypeStruct((EMBED_DIM, BATCH), jnp.float32),
        mesh=mesh,
        scratch_types=[
            pltpu.VMEM((b_per_w,), jnp.int32),
            pltpu.VMEM((b_per_w,), jnp.int32),
            pltpu.VMEM((EMBED_DIM, n_chunks, ICHUNK), jnp.int32),
            pltpu.VMEM((EMBED_DIM, n_chunks, ICHUNK), jnp.int32),
            pltpu.VMEM((EMBED_DIM, n_chunks, ICHUNK), jnp.float32),
            pltpu.VMEM((EMBED_DIM, n_chunks, ICHUNK), jnp.float32),
            pltpu.VMEM((EMBED_DIM, b_per_w), jnp.float32),
            pltpu.SemaphoreType.DMA,
            pltpu.SemaphoreType.DMA,
        ],
    )
    def gmf(user_hbm, item_hbm, ut_hbm, it_hbm, out_hbm,
            uidx_v, iidx_v, ueidx_v, ieidx_v, ug_v, ig_v, out_v,
            sem_u, sem_i):
        wid = lax.axis_index("s") * nc + lax.axis_index("c")
        base = wid * b_per_w
        pltpu.sync_copy(user_hbm.at[pl.ds(base, b_per_w)], uidx_v)
        pltpu.sync_copy(item_hbm.at[pl.ds(base, b_per_w)], iidx_v)

        for j in range(EMBED_DIM):
            def build(v, _):
                c = lax.shift_right_logical(v, 3)
                o = lax.mul(lax.bitwise_and(v, 7), VEC)
                uvec = uidx_v[pl.ds(lax.mul(v, VEC), VEC)]
                ivec = iidx_v[pl.ds(lax.mul(v, VEC), VEC)]
                ueidx_v[j, c, pl.ds(o, VEC)] = uvec + j * NUM_ROWS
                ieidx_v[j, c, pl.ds(o, VEC)] = ivec + j * NUM_ROWS
                return 0
            lax.fori_loop(0, b_per_w // VEC, build, 0)

        copies = []
        for j in range(EMBED_DIM):
            for c in range(n_chunks):
                copies.append(pltpu.async_copy(
                    ut_hbm.at[ueidx_v.at[j, c]], ug_v.at[j, c], sem_u))
                copies.append(pltpu.async_copy(
                    it_hbm.at[ieidx_v.at[j, c]], ig_v.at[j, c], sem_i))
        for cp in copies:
            cp.wait()

        for j in range(EMBED_DIM):
            def mul(v, _):
                c = lax.shift_right_logical(v, 3)
                o = lax.mul(lax.bitwise_and(v, 7), VEC)
                out_v[j, pl.ds(lax.mul(v, VEC), VEC)] = (
                    ug_v[j, c, pl.ds(o, VEC)] * ig_v[j, c, pl.ds(o, VEC)])
                return 0
            lax.fori_loop(0, b_per_w // VEC, mul, 0)

        pltpu.sync_copy(out_v, out_hbm.at[:, pl.ds(base, b_per_w)])

    out_t = gmf(user, item, ut1, it1)
    return out_t.T


def kernel(user, item, user_table, item_table):
    return _gmf(user, item, user_table, item_table)


# speedup vs baseline: 1.0024x; 1.0024x over previous
import functools

import jax
import jax.numpy as jnp
from jax import lax
from jax.experimental import pallas as pl
from jax.experimental.pallas import tpu as pltpu
from jax.experimental.pallas import tpu_sc as plsc

NUM_ROWS = 1000000
BATCH = 16384
EMBED_DIM = 16
VEC = 16
ICHUNK = 128


@jax.jit
def _gmf(user, item, user_table, item_table):
    info = plsc.get_sparse_core_info()
    nc, ns = info.num_cores, info.num_subcores
    nw = nc * ns
    b_per_w = BATCH // nw
    n_chunks = b_per_w // ICHUNK

    utT = user_table.T
    itT = item_table.T

    mesh = plsc.VectorSubcoreMesh(core_axis_name="c", subcore_axis_name="s")

    @functools.partial(
        pl.kernel,
        out_type=jax.ShapeDtypeStruct((EMBED_DIM, BATCH), jnp.float32),
        mesh=mesh,
        compiler_params=pltpu.CompilerParams(use_tc_tiling_on_sc=False),
        scratch_types=[
            pltpu.VMEM((n_chunks, ICHUNK), jnp.int32),
            pltpu.VMEM((n_chunks, ICHUNK), jnp.int32),
            pltpu.VMEM((EMBED_DIM, n_chunks, ICHUNK), jnp.float32),
            pltpu.VMEM((EMBED_DIM, n_chunks, ICHUNK), jnp.float32),
            pltpu.VMEM((EMBED_DIM, b_per_w), jnp.float32),
            pltpu.SemaphoreType.DMA,
            pltpu.SemaphoreType.DMA,
        ],
    )
    def gmf(user_hbm, item_hbm, ut_hbm, it_hbm, out_hbm,
            uidx_v, iidx_v, ug_v, ig_v, out_v, sem_u, sem_i):
        wid = lax.axis_index("s") * nc + lax.axis_index("c")
        base = wid * b_per_w
        for c in range(n_chunks):
            pltpu.sync_copy(
                user_hbm.at[pl.ds(base + c * ICHUNK, ICHUNK)], uidx_v.at[c])
            pltpu.sync_copy(
                item_hbm.at[pl.ds(base + c * ICHUNK, ICHUNK)], iidx_v.at[c])

        copies = []
        for j in range(EMBED_DIM):
            for c in range(n_chunks):
                copies.append(pltpu.async_copy(
                    ut_hbm.at[j].at[uidx_v.at[c]], ug_v.at[j, c], sem_u))
                copies.append(pltpu.async_copy(
                    it_hbm.at[j].at[iidx_v.at[c]], ig_v.at[j, c], sem_i))
        for cp in copies:
            cp.wait()

        for j in range(EMBED_DIM):
            def mul(v, _):
                c = lax.shift_right_logical(v, 3)
                o = lax.mul(lax.bitwise_and(v, 7), VEC)
                out_v[j, pl.ds(lax.mul(v, VEC), VEC)] = (
                    ug_v[j, c, pl.ds(o, VEC)] * ig_v[j, c, pl.ds(o, VEC)])
                return 0
            lax.fori_loop(0, b_per_w // VEC, mul, 0)

        pltpu.sync_copy(out_v, out_hbm.at[:, pl.ds(base, b_per_w)])

    out_t = gmf(user, item, utT, itT)
    return out_t.T


def kernel(user, item, user_table, item_table):
    return _gmf(user, item, user_table, item_table)


# speedup vs baseline: 18.9192x; 18.8746x over previous
import functools

import jax
import jax.numpy as jnp
from jax import lax
from jax.experimental import pallas as pl
from jax.experimental.pallas import tpu as pltpu
from jax.experimental.pallas import tpu_sc as plsc

NUM_ROWS = 1000000
BATCH = 16384
EMBED_DIM = 16
VEC = 16
ICHUNK = 128
CB = 32768
NB = -(-NUM_ROWS // CB)
TAIL = NUM_ROWS - (NB - 1) * CB
TAILP = -(-TAIL // 128) * 128
STRIDE = -(-NUM_ROWS // 128) * 128


def _detile_body(ut_ref, it_ref, ul_ref, il_ref, sem):
    c = pl.program_id(0)
    base = c * CB

    def emit(width):
        copies = []
        for j in range(EMBED_DIM):
            copies.append(pltpu.async_copy(
                ut_ref.at[j, pl.ds(0, width)],
                ul_ref.at[pl.ds(j * STRIDE + base, width)], sem))
            copies.append(pltpu.async_copy(
                it_ref.at[j, pl.ds(0, width)],
                il_ref.at[pl.ds(j * STRIDE + base, width)], sem))
        for cp in copies:
            cp.wait()

    @pl.when(c < NB - 1)
    def _():
        emit(CB)

    @pl.when(c == NB - 1)
    def _():
        emit(TAILP)


@jax.jit
def _gmf(user, item, user_table, item_table):
    info = plsc.get_sparse_core_info()
    nc, ns = info.num_cores, info.num_subcores
    nw = nc * ns
    b_per_w = BATCH // nw
    n_chunks = b_per_w // ICHUNK

    utT = user_table.T
    itT = item_table.T

    ulin, ilin = pl.pallas_call(
        _detile_body,
        grid=(NB,),
        in_specs=[
            pl.BlockSpec((EMBED_DIM, CB), lambda c: (0, c)),
            pl.BlockSpec((EMBED_DIM, CB), lambda c: (0, c)),
        ],
        out_specs=[
            pl.BlockSpec(memory_space=pl.ANY),
            pl.BlockSpec(memory_space=pl.ANY),
        ],
        out_shape=[
            jax.ShapeDtypeStruct((STRIDE * EMBED_DIM,), jnp.float32),
            jax.ShapeDtypeStruct((STRIDE * EMBED_DIM,), jnp.float32),
        ],
        scratch_shapes=[pltpu.SemaphoreType.DMA],
        compiler_params=pltpu.CompilerParams(
            dimension_semantics=("arbitrary",)),
    )(utT, itT)

    mesh = plsc.VectorSubcoreMesh(core_axis_name="c", subcore_axis_name="s")

    @functools.partial(
        pl.kernel,
        out_type=jax.ShapeDtypeStruct((EMBED_DIM, BATCH), jnp.float32),
        mesh=mesh,
        scratch_types=[
            pltpu.VMEM((n_chunks, ICHUNK), jnp.int32),
            pltpu.VMEM((n_chunks, ICHUNK), jnp.int32),
            pltpu.VMEM((EMBED_DIM, n_chunks, ICHUNK), jnp.float32),
            pltpu.VMEM((EMBED_DIM, n_chunks, ICHUNK), jnp.float32),
            pltpu.VMEM((EMBED_DIM, b_per_w), jnp.float32),
            pltpu.SemaphoreType.DMA,
            pltpu.SemaphoreType.DMA,
        ],
    )
    def gmf(user_hbm, item_hbm, ul_hbm, il_hbm, out_hbm,
            uidx_v, iidx_v, ug_v, ig_v, out_v, sem_u, sem_i):
        wid = lax.axis_index("s") * nc + lax.axis_index("c")
        base = wid * b_per_w
        for c in range(n_chunks):
            pltpu.sync_copy(
                user_hbm.at[pl.ds(base + c * ICHUNK, ICHUNK)], uidx_v.at[c])
            pltpu.sync_copy(
                item_hbm.at[pl.ds(base + c * ICHUNK, ICHUNK)], iidx_v.at[c])

        copies = []
        for j in range(EMBED_DIM):
            for c in range(n_chunks):
                copies.append(pltpu.async_copy(
                    ul_hbm.at[pl.ds(j * STRIDE, NUM_ROWS)].at[uidx_v.at[c]],
                    ug_v.at[j, c], sem_u))
                copies.append(pltpu.async_copy(
                    il_hbm.at[pl.ds(j * STRIDE, NUM_ROWS)].at[iidx_v.at[c]],
                    ig_v.at[j, c], sem_i))
        for cp in copies:
            cp.wait()

        for j in range(EMBED_DIM):
            def mul(v, _):
                c = lax.shift_right_logical(v, 3)
                o = lax.mul(lax.bitwise_and(v, 7), VEC)
                out_v[j, pl.ds(lax.mul(v, VEC), VEC)] = (
                    ug_v[j, c, pl.ds(o, VEC)] * ig_v[j, c, pl.ds(o, VEC)])
                return 0
            lax.fori_loop(0, b_per_w // VEC, mul, 0)

        pltpu.sync_copy(out_v, out_hbm.at[:, pl.ds(base, b_per_w)])

    out_t = gmf(user, item, ulin, ilin)
    return out_t.T


def kernel(user, item, user_table, item_table):
    return _gmf(user, item, user_table, item_table)


# speedup vs baseline: 20.5993x; 1.0888x over previous
import functools

import jax
import jax.numpy as jnp
from jax import lax
from jax.experimental import pallas as pl
from jax.experimental.pallas import tpu as pltpu
from jax.experimental.pallas import tpu_sc as plsc

NUM_ROWS = 1000000
BATCH = 16384
EMBED_DIM = 16
VEC = 16
ICHUNK = 128
CB = 65536
NB = -(-NUM_ROWS // CB)
TAIL = NUM_ROWS - (NB - 1) * CB
TAILP = -(-TAIL // 128) * 128
STRIDE = -(-NUM_ROWS // 128) * 128


def _detile_body(ut_ref, it_ref, ul_ref, il_ref, sem):
    c = pl.program_id(0)
    base = c * CB

    def emit(width):
        copies = []
        for j in range(EMBED_DIM):
            copies.append(pltpu.async_copy(
                ut_ref.at[j, pl.ds(0, width)],
                ul_ref.at[pl.ds(j * STRIDE + base, width)], sem))
            copies.append(pltpu.async_copy(
                it_ref.at[j, pl.ds(0, width)],
                il_ref.at[pl.ds(j * STRIDE + base, width)], sem))
        for cp in copies:
            cp.wait()

    @pl.when(c < NB - 1)
    def _():
        emit(CB)

    @pl.when(c == NB - 1)
    def _():
        emit(TAILP)


@jax.jit
def _gmf(user, item, user_table, item_table):
    info = plsc.get_sparse_core_info()
    nc, ns = info.num_cores, info.num_subcores
    nw = nc * ns
    b_per_w = BATCH // nw
    n_chunks = b_per_w // ICHUNK

    utT = user_table.T
    itT = item_table.T

    ulin, ilin = pl.pallas_call(
        _detile_body,
        grid=(NB,),
        in_specs=[
            pl.BlockSpec((EMBED_DIM, CB), lambda c: (0, c)),
            pl.BlockSpec((EMBED_DIM, CB), lambda c: (0, c)),
        ],
        out_specs=[
            pl.BlockSpec(memory_space=pl.ANY),
            pl.BlockSpec(memory_space=pl.ANY),
        ],
        out_shape=[
            jax.ShapeDtypeStruct((STRIDE * EMBED_DIM,), jnp.float32),
            jax.ShapeDtypeStruct((STRIDE * EMBED_DIM,), jnp.float32),
        ],
        scratch_shapes=[pltpu.SemaphoreType.DMA],
        compiler_params=pltpu.CompilerParams(
            dimension_semantics=("arbitrary",)),
    )(utT, itT)

    mesh = plsc.VectorSubcoreMesh(core_axis_name="c", subcore_axis_name="s")

    @functools.partial(
        pl.kernel,
        out_type=jax.ShapeDtypeStruct((EMBED_DIM, BATCH), jnp.float32),
        mesh=mesh,
        scratch_types=[
            pltpu.VMEM((n_chunks, ICHUNK), jnp.int32),
            pltpu.VMEM((n_chunks, ICHUNK), jnp.int32),
            pltpu.VMEM((EMBED_DIM, n_chunks, ICHUNK), jnp.float32),
            pltpu.VMEM((EMBED_DIM, n_chunks, ICHUNK), jnp.float32),
            pltpu.VMEM((EMBED_DIM, b_per_w), jnp.float32),
            pltpu.SemaphoreType.DMA,
            pltpu.SemaphoreType.DMA,
        ],
    )
    def gmf(user_hbm, item_hbm, ul_hbm, il_hbm, out_hbm,
            uidx_v, iidx_v, ug_v, ig_v, out_v, sem_u, sem_i):
        wid = lax.axis_index("s") * nc + lax.axis_index("c")
        base = wid * b_per_w
        for c in range(n_chunks):
            pltpu.sync_copy(
                user_hbm.at[pl.ds(base + c * ICHUNK, ICHUNK)], uidx_v.at[c])
            pltpu.sync_copy(
                item_hbm.at[pl.ds(base + c * ICHUNK, ICHUNK)], iidx_v.at[c])

        copies = []
        for j in range(EMBED_DIM):
            for c in range(n_chunks):
                copies.append(pltpu.async_copy(
                    ul_hbm.at[pl.ds(j * STRIDE, NUM_ROWS)].at[uidx_v.at[c]],
                    ug_v.at[j, c], sem_u))
                copies.append(pltpu.async_copy(
                    il_hbm.at[pl.ds(j * STRIDE, NUM_ROWS)].at[iidx_v.at[c]],
                    ig_v.at[j, c], sem_i))
        for cp in copies:
            cp.wait()

        for j in range(EMBED_DIM):
            def mul(v, _):
                c = lax.shift_right_logical(v, 3)
                o = lax.mul(lax.bitwise_and(v, 7), VEC)
                out_v[j, pl.ds(lax.mul(v, VEC), VEC)] = (
                    ug_v[j, c, pl.ds(o, VEC)] * ig_v[j, c, pl.ds(o, VEC)])
                return 0
            lax.fori_loop(0, b_per_w // VEC, mul, 0)

        pltpu.sync_copy(out_v, out_hbm.at[:, pl.ds(base, b_per_w)])

    out_t = gmf(user, item, ulin, ilin)
    return out_t.T


def kernel(user, item, user_table, item_table):
    return _gmf(user, item, user_table, item_table)


# speedup vs baseline: 20.9699x; 1.0180x over previous
import functools

import jax
import jax.numpy as jnp
from jax import lax
from jax.experimental import pallas as pl
from jax.experimental.pallas import tpu as pltpu
from jax.experimental.pallas import tpu_sc as plsc

NUM_ROWS = 1000000
BATCH = 16384
EMBED_DIM = 16
VEC = 16
ICHUNK = 128
CB = 131072
NB = -(-NUM_ROWS // CB)
TAIL = NUM_ROWS - (NB - 1) * CB
TAILP = -(-TAIL // 128) * 128
STRIDE = -(-NUM_ROWS // 128) * 128


def _detile_body(ut_ref, it_ref, ul_ref, il_ref, sem):
    c = pl.program_id(0)
    base = c * CB

    def emit(width):
        copies = []
        for j in range(EMBED_DIM):
            copies.append(pltpu.async_copy(
                ut_ref.at[j, pl.ds(0, width)],
                ul_ref.at[pl.ds(j * STRIDE + base, width)], sem))
            copies.append(pltpu.async_copy(
                it_ref.at[j, pl.ds(0, width)],
                il_ref.at[pl.ds(j * STRIDE + base, width)], sem))
        for cp in copies:
            cp.wait()

    @pl.when(c < NB - 1)
    def _():
        emit(CB)

    @pl.when(c == NB - 1)
    def _():
        emit(TAILP)


@jax.jit
def _gmf(user, item, user_table, item_table):
    info = plsc.get_sparse_core_info()
    nc, ns = info.num_cores, info.num_subcores
    nw = nc * ns
    b_per_w = BATCH // nw
    n_chunks = b_per_w // ICHUNK

    utT = user_table.T
    itT = item_table.T

    ulin, ilin = pl.pallas_call(
        _detile_body,
        grid=(NB,),
        in_specs=[
            pl.BlockSpec((EMBED_DIM, CB), lambda c: (0, c)),
            pl.BlockSpec((EMBED_DIM, CB), lambda c: (0, c)),
        ],
        out_specs=[
            pl.BlockSpec(memory_space=pl.ANY),
            pl.BlockSpec(memory_space=pl.ANY),
        ],
        out_shape=[
            jax.ShapeDtypeStruct((STRIDE * EMBED_DIM,), jnp.float32),
            jax.ShapeDtypeStruct((STRIDE * EMBED_DIM,), jnp.float32),
        ],
        scratch_shapes=[pltpu.SemaphoreType.DMA],
        compiler_params=pltpu.CompilerParams(
            dimension_semantics=("arbitrary",)),
    )(utT, itT)

    mesh = plsc.VectorSubcoreMesh(core_axis_name="c", subcore_axis_name="s")

    @functools.partial(
        pl.kernel,
        out_type=jax.ShapeDtypeStruct((EMBED_DIM, BATCH), jnp.float32),
        mesh=mesh,
        scratch_types=[
            pltpu.VMEM((n_chunks, ICHUNK), jnp.int32),
            pltpu.VMEM((n_chunks, ICHUNK), jnp.int32),
            pltpu.VMEM((EMBED_DIM, n_chunks, ICHUNK), jnp.float32),
            pltpu.VMEM((EMBED_DIM, n_chunks, ICHUNK), jnp.float32),
            pltpu.VMEM((EMBED_DIM, b_per_w), jnp.float32),
            pltpu.SemaphoreType.DMA,
            pltpu.SemaphoreType.DMA,
        ],
    )
    def gmf(user_hbm, item_hbm, ul_hbm, il_hbm, out_hbm,
            uidx_v, iidx_v, ug_v, ig_v, out_v, sem_u, sem_i):
        wid = lax.axis_index("s") * nc + lax.axis_index("c")
        base = wid * b_per_w
        for c in range(n_chunks):
            pltpu.sync_copy(
                user_hbm.at[pl.ds(base + c * ICHUNK, ICHUNK)], uidx_v.at[c])
            pltpu.sync_copy(
                item_hbm.at[pl.ds(base + c * ICHUNK, ICHUNK)], iidx_v.at[c])

        copies = []
        for j in range(EMBED_DIM):
            for c in range(n_chunks):
                copies.append(pltpu.async_copy(
                    ul_hbm.at[pl.ds(j * STRIDE, NUM_ROWS)].at[uidx_v.at[c]],
                    ug_v.at[j, c], sem_u))
                copies.append(pltpu.async_copy(
                    il_hbm.at[pl.ds(j * STRIDE, NUM_ROWS)].at[iidx_v.at[c]],
                    ig_v.at[j, c], sem_i))
        for cp in copies:
            cp.wait()

        for j in range(EMBED_DIM):
            def mul(v, _):
                c = lax.shift_right_logical(v, 3)
                o = lax.mul(lax.bitwise_and(v, 7), VEC)
                out_v[j, pl.ds(lax.mul(v, VEC), VEC)] = (
                    ug_v[j, c, pl.ds(o, VEC)] * ig_v[j, c, pl.ds(o, VEC)])
                return 0
            lax.fori_loop(0, b_per_w // VEC, mul, 0)

        pltpu.sync_copy(out_v, out_hbm.at[:, pl.ds(base, b_per_w)])

    out_t = gmf(user, item, ulin, ilin)
    return out_t.T


def kernel(user, item, user_table, item_table):
    return _gmf(user, item, user_table, item_table)


# speedup vs baseline: 21.9896x; 1.0486x over previous
import functools

import jax
import jax.numpy as jnp
from jax import lax
from jax.experimental import pallas as pl
from jax.experimental.pallas import tpu as pltpu
from jax.experimental.pallas import tpu_sc as plsc

NUM_ROWS = 1000000
BATCH = 16384
EMBED_DIM = 16
NPAIR = EMBED_DIM // 2
VEC = 16
ICHUNK = 128
CB = 131072
NB = -(-NUM_ROWS // CB)
TAIL = NUM_ROWS - (NB - 1) * CB
TAILP = -(-TAIL // 128) * 128
STRIDE = -(-NUM_ROWS // 128) * 128


def _detile_body(ut_ref, it_ref, ul_ref, il_ref, up_ref, ip_ref, sem):
    c = pl.program_id(0)
    base = c * CB

    def pack(src_ref, dst_ref):
        x16 = lax.bitcast_convert_type(
            src_ref[...].astype(jnp.bfloat16), jnp.uint16)
        for jp in range(NPAIR):
            lo = x16[2 * jp:2 * jp + 1, :].astype(jnp.uint32)
            hi = x16[2 * jp + 1:2 * jp + 2, :].astype(jnp.uint32)
            dst_ref[jp:jp + 1, :] = lo | (hi << 16)

    pack(ut_ref, up_ref)
    pack(it_ref, ip_ref)

    def emit(width):
        copies = []
        for jp in range(NPAIR):
            copies.append(pltpu.async_copy(
                up_ref.at[jp, pl.ds(0, width)],
                ul_ref.at[pl.ds(jp * STRIDE + base, width)], sem))
            copies.append(pltpu.async_copy(
                ip_ref.at[jp, pl.ds(0, width)],
                il_ref.at[pl.ds(jp * STRIDE + base, width)], sem))
        for cp in copies:
            cp.wait()

    @pl.when(c < NB - 1)
    def _():
        emit(CB)

    @pl.when(c == NB - 1)
    def _():
        emit(TAILP)


@jax.jit
def _gmf(user, item, user_table, item_table):
    info = plsc.get_sparse_core_info()
    nc, ns = info.num_cores, info.num_subcores
    nw = nc * ns
    b_per_w = BATCH // nw
    n_chunks = b_per_w // ICHUNK

    utT = user_table.T
    itT = item_table.T

    ulin, ilin = pl.pallas_call(
        _detile_body,
        grid=(NB,),
        in_specs=[
            pl.BlockSpec((EMBED_DIM, CB), lambda c: (0, c)),
            pl.BlockSpec((EMBED_DIM, CB), lambda c: (0, c)),
        ],
        out_specs=[
            pl.BlockSpec(memory_space=pl.ANY),
            pl.BlockSpec(memory_space=pl.ANY),
        ],
        out_shape=[
            jax.ShapeDtypeStruct((STRIDE * NPAIR,), jnp.uint32),
            jax.ShapeDtypeStruct((STRIDE * NPAIR,), jnp.uint32),
        ],
        scratch_shapes=[
            pltpu.VMEM((NPAIR, CB), jnp.uint32),
            pltpu.VMEM((NPAIR, CB), jnp.uint32),
            pltpu.SemaphoreType.DMA,
        ],
        compiler_params=pltpu.CompilerParams(
            dimension_semantics=("arbitrary",)),
    )(utT, itT)

    mesh = plsc.VectorSubcoreMesh(core_axis_name="c", subcore_axis_name="s")

    @functools.partial(
        pl.kernel,
        out_type=jax.ShapeDtypeStruct((NPAIR, BATCH), jnp.uint32),
        mesh=mesh,
        compiler_params=pltpu.CompilerParams(needs_layout_passes=False),
        scratch_types=[
            pltpu.VMEM((n_chunks, ICHUNK), jnp.int32),
            pltpu.VMEM((n_chunks, ICHUNK), jnp.int32),
            pltpu.VMEM((NPAIR, n_chunks, ICHUNK), jnp.uint32),
            pltpu.VMEM((NPAIR, n_chunks, ICHUNK), jnp.uint32),
            pltpu.VMEM((NPAIR, b_per_w), jnp.uint32),
            pltpu.SemaphoreType.DMA,
            pltpu.SemaphoreType.DMA,
        ],
    )
    def gmf(user_hbm, item_hbm, ul_hbm, il_hbm, out_hbm,
            uidx_v, iidx_v, ug_v, ig_v, out_v, sem_u, sem_i):
        wid = lax.axis_index("s") * nc + lax.axis_index("c")
        base = wid * b_per_w
        for c in range(n_chunks):
            pltpu.sync_copy(
                user_hbm.at[pl.ds(base + c * ICHUNK, ICHUNK)], uidx_v.at[c])
            pltpu.sync_copy(
                item_hbm.at[pl.ds(base + c * ICHUNK, ICHUNK)], iidx_v.at[c])

        copies = []
        for jp in range(NPAIR):
            for c in range(n_chunks):
                copies.append(pltpu.async_copy(
                    ul_hbm.at[pl.ds(jp * STRIDE, NUM_ROWS)].at[uidx_v.at[c]],
                    ug_v.at[jp, c], sem_u))
                copies.append(pltpu.async_copy(
                    il_hbm.at[pl.ds(jp * STRIDE, NUM_ROWS)].at[iidx_v.at[c]],
                    ig_v.at[jp, c], sem_i))
        for cp in copies:
            cp.wait()

        for jp in range(NPAIR):
            def mul(v, _):
                c = lax.shift_right_logical(v, 3)
                o = lax.mul(lax.bitwise_and(v, 7), VEC)
                ub = plsc.bitcast(ug_v[jp, c, pl.ds(o, VEC)], jnp.bfloat16)
                ib = plsc.bitcast(ig_v[jp, c, pl.ds(o, VEC)], jnp.bfloat16)
                out_v[jp, pl.ds(lax.mul(v, VEC), VEC)] = plsc.bitcast(
                    ub * ib, jnp.uint32)
                return 0
            lax.fori_loop(0, b_per_w // VEC, mul, 0)

        pltpu.sync_copy(out_v, out_hbm.at[:, pl.ds(base, b_per_w)])

    out_pairs = gmf(user, item, ulin, ilin)
    out_bf = lax.bitcast_convert_type(out_pairs, jnp.bfloat16)
    return out_bf.transpose(1, 0, 2).reshape(BATCH, EMBED_DIM).astype(
        jnp.float32)


def kernel(user, item, user_table, item_table):
    return _gmf(user, item, user_table, item_table)
